# baseline (speedup 1.0000x reference)
import functools

import jax
import jax.numpy as jnp
from jax import lax
from jax.experimental import pallas as pl
from jax.experimental.pallas import tpu as pltpu
from jax.experimental.pallas import tpu_sc as plsc

N = 100000
D = 128
PICK = 12
KPAD = 16
BN = 20000
GRID = N // BN
NEG = -1e30
IMAX = 2147483647
_DOT_NT = (((1,), (1,)), ((), ()))
_DOT_NN = (((1,), (0,)), ((), ()))


def _sc_gather_rows(embed, idx16):
    mesh = plsc.VectorSubcoreMesh(core_axis_name="c", subcore_axis_name="s")

    @functools.partial(
        pl.kernel,
        out_type=jax.ShapeDtypeStruct((KPAD, D), jnp.float32),
        mesh=mesh,
        scratch_types=[
            pltpu.VMEM((KPAD,), jnp.int32),
            pltpu.VMEM((KPAD, D), jnp.float32),
            pltpu.SemaphoreType.DMA,
        ],
    )
    def gather_kernel(embed_hbm, idx_hbm, out_hbm, idx_v, rows_v, sem):
        c = lax.axis_index("c")
        s = lax.axis_index("s")

        @pl.when(jnp.logical_and(c == 0, s == 0))
        def _():
            pltpu.sync_copy(idx_hbm, idx_v)
            pltpu.async_copy(embed_hbm.at[idx_v], rows_v, sem).wait()
            pltpu.sync_copy(rows_v, out_hbm)

    return gather_kernel(embed, idx16)


def _pass1_topk(embed, rows_s, Wq, Wk, bq_col, bk_row):
    def body(embed_ref, rows_ref, wq_ref, wk_ref, bqc_ref, bkr_ref,
             idx_ref, qa_ref, c_ref, cv_ref, ci_ref):
        i = pl.program_id(0)

        @pl.when(i == 0)
        def _():
            nk = lax.dot_general(rows_ref[...], wk_ref[...], _DOT_NT,
                                 preferred_element_type=jnp.float32) + bkr_ref[...]
            qa_ref[...] = lax.dot_general(nk, wq_ref[...], _DOT_NN,
                                          preferred_element_type=jnp.float32)
            cc = lax.dot_general(nk, bqc_ref[...], _DOT_NN,
                                 preferred_element_type=jnp.float32)
            rid = lax.broadcasted_iota(jnp.int32, (KPAD, 1), 0)
            c_ref[...] = jnp.where(rid >= PICK, NEG, cc)

        st = lax.dot_general(qa_ref[...], embed_ref[...], _DOT_NT,
                             preferred_element_type=jnp.float32)
        mvb = jnp.max(st + c_ref[...], axis=0, keepdims=True)

        gidx = lax.broadcasted_iota(jnp.int32, (1, BN), 1) + i * BN
        lane16 = lax.broadcasted_iota(jnp.int32, (1, KPAD), 1)
        cv = jnp.full((1, KPAD), NEG, jnp.float32)
        ci = jnp.full((1, KPAD), IMAX, jnp.int32)
        for t in range(PICK):
            m = jnp.max(mvb, axis=1, keepdims=True)
            sel = jnp.min(jnp.where(mvb >= m, gidx, IMAX),
                          axis=1, keepdims=True)
            mvb = jnp.where(gidx == sel, NEG, mvb)
            cv = jnp.where(lane16 == t, m, cv)
            ci = jnp.where(lane16 == t, sel, ci)
        for j in range(GRID):
            @pl.when(i == j)
            def _(cv=cv, ci=ci, j=j):
                cv_ref[j, :] = cv[0, :]
                ci_ref[j, :] = ci[0, :]

        @pl.when(i == GRID - 1)
        def _():
            cV = cv_ref[...]
            cI = ci_ref[...]
            for t in range(PICK):
                m = jnp.max(cV)
                sel = jnp.min(jnp.where(cV >= m, cI, IMAX))
                cV = jnp.where((cV >= m) & (cI == sel), NEG, cV)
                idx_ref[t] = sel
            for t in range(PICK, KPAD):
                idx_ref[t] = 0

    return pl.pallas_call(
        body,
        grid=(GRID,),
        in_specs=[
            pl.BlockSpec((BN, D), lambda i: (i, 0)),
            pl.BlockSpec((KPAD, D), lambda i: (0, 0)),
            pl.BlockSpec((D, D), lambda i: (0, 0)),
            pl.BlockSpec((D, D), lambda i: (0, 0)),
            pl.BlockSpec((D, 1), lambda i: (0, 0)),
            pl.BlockSpec((1, D), lambda i: (0, 0)),
        ],
        out_specs=pl.BlockSpec(memory_space=pltpu.SMEM),
        out_shape=jax.ShapeDtypeStruct((KPAD,), jnp.int32),
        scratch_shapes=[
            pltpu.VMEM((KPAD, D), jnp.float32),
            pltpu.VMEM((KPAD, 1), jnp.float32),
            pltpu.VMEM((GRID, KPAD), jnp.float32),
            pltpu.VMEM((GRID, KPAD), jnp.int32),
        ],
    )(embed, rows_s, Wq, Wk, bq_col, bk_row)


def _pass2_pool(embed, rows_t, Wq, Wk, bq_row, bk_col):
    def body(embed_ref, rows_ref, wq_ref, wk_ref, bqr_ref, bkc_ref,
             out_ref, b_ref, d_ref):
        i = pl.program_id(0)

        @pl.when(i == 0)
        def _():
            qr = lax.dot_general(rows_ref[...], wq_ref[...], _DOT_NT,
                                 preferred_element_type=jnp.float32) + bqr_ref[...]
            b_ref[...] = lax.dot_general(qr, wk_ref[...], _DOT_NN,
                                         preferred_element_type=jnp.float32)
            dd = lax.dot_general(qr, bkc_ref[...], _DOT_NN,
                                 preferred_element_type=jnp.float32)
            rid = lax.broadcasted_iota(jnp.int32, (KPAD, 1), 0)
            d_ref[...] = jnp.where(rid >= PICK, NEG, dd)

        tt = lax.dot_general(b_ref[...], embed_ref[...], _DOT_NT,
                             preferred_element_type=jnp.float32)
        p = jnp.max(tt + d_ref[...], axis=0, keepdims=True)
        contrib = lax.dot_general(p, embed_ref[...], _DOT_NN,
                                  preferred_element_type=jnp.float32)

        @pl.when(i == 0)
        def _():
            out_ref[...] = contrib

        @pl.when(i > 0)
        def _():
            out_ref[...] = out_ref[...] + contrib

    return pl.pallas_call(
        body,
        grid=(GRID,),
        in_specs=[
            pl.BlockSpec((BN, D), lambda i: (i, 0)),
            pl.BlockSpec((KPAD, D), lambda i: (0, 0)),
            pl.BlockSpec((D, D), lambda i: (0, 0)),
            pl.BlockSpec((D, D), lambda i: (0, 0)),
            pl.BlockSpec((1, D), lambda i: (0, 0)),
            pl.BlockSpec((D, 1), lambda i: (0, 0)),
        ],
        out_specs=pl.BlockSpec((1, D), lambda i: (0, 0)),
        out_shape=jax.ShapeDtypeStruct((1, D), jnp.float32),
        scratch_shapes=[
            pltpu.VMEM((KPAD, D), jnp.float32),
            pltpu.VMEM((KPAD, 1), jnp.float32),
        ],
    )(embed, rows_t, Wq, Wk, bq_row, bk_col)


def kernel(embed_matrix, Wq, bq, Wk, bk, sample_indices):
    idx16 = jnp.concatenate(
        [sample_indices.astype(jnp.int32),
         jnp.zeros((KPAD - PICK,), jnp.int32)])
    rows_s = _sc_gather_rows(embed_matrix, idx16)
    top_idx = _pass1_topk(embed_matrix, rows_s, Wq, Wk,
                          bq.reshape(D, 1), bk.reshape(1, D))
    rows_t = _sc_gather_rows(embed_matrix, top_idx)
    return _pass2_pool(embed_matrix, rows_t, Wq, Wk,
                       bq.reshape(1, D), bk.reshape(D, 1))


# speedup vs baseline: 1.4415x; 1.4415x over previous
import functools

import jax
import jax.numpy as jnp
from jax import lax
from jax.experimental import pallas as pl
from jax.experimental.pallas import tpu as pltpu
from jax.experimental.pallas import tpu_sc as plsc

N = 100000
D = 128
PICK = 12
KPAD = 16
BN = 10000
GRID = N // BN
NEG = -1e30
IMAX = 2147483647
_DOT_NT = (((1,), (1,)), ((), ()))
_DOT_NN = (((1,), (0,)), ((), ()))


def _sc_gather_rows(embed, idx16):
    mesh = plsc.VectorSubcoreMesh(core_axis_name="c", subcore_axis_name="s")

    @functools.partial(
        pl.kernel,
        out_type=jax.ShapeDtypeStruct((KPAD, D), jnp.float32),
        mesh=mesh,
        scratch_types=[
            pltpu.VMEM((KPAD,), jnp.int32),
            pltpu.VMEM((KPAD, D), jnp.float32),
            pltpu.SemaphoreType.DMA,
        ],
    )
    def gather_kernel(embed_hbm, idx_hbm, out_hbm, idx_v, rows_v, sem):
        c = lax.axis_index("c")
        s = lax.axis_index("s")

        @pl.when(jnp.logical_and(c == 0, s == 0))
        def _():
            pltpu.sync_copy(idx_hbm, idx_v)
            pltpu.async_copy(embed_hbm.at[idx_v], rows_v, sem).wait()
            pltpu.sync_copy(rows_v, out_hbm)

    return gather_kernel(embed, idx16)


def _fused_passes(embed, rows_s, Wq, Wk, bq_col, bq_row, bk_row, bk_col):
    def body(embed_ref, embed_any, rows_ref, wq_ref, wk_ref, bqc_ref,
             bqr_ref, bkr_ref, bkc_ref, out_ref, qa_ref, c_ref, b16_ref,
             d_ref, mv_ref, ebb_ref, rows2_ref, sem_row):
        i = pl.program_id(0)

        @pl.when(i == 0)
        def _():
            nk = lax.dot_general(rows_ref[...], wk_ref[...], _DOT_NT,
                                 preferred_element_type=jnp.float32) + bkr_ref[...]
            qa_ref[...] = lax.dot_general(nk, wq_ref[...], _DOT_NN,
                                          preferred_element_type=jnp.float32)
            cc = lax.dot_general(nk, bqc_ref[...], _DOT_NN,
                                 preferred_element_type=jnp.float32)
            rid = lax.broadcasted_iota(jnp.int32, (KPAD, 1), 0)
            c_ref[...] = jnp.where(rid >= PICK, NEG, cc)
            rows2_ref[...] = jnp.zeros((KPAD, D), jnp.float32)

        @pl.when(i < GRID)
        def _():
            blk = embed_ref[...]
            st = lax.dot_general(qa_ref[...], blk, _DOT_NT,
                                 preferred_element_type=jnp.float32)
            mvb = jnp.max(st + c_ref[...], axis=0, keepdims=True)
            for j in range(GRID):
                @pl.when(i == j)
                def _(blk=blk, mvb=mvb, j=j):
                    ebb_ref[j * BN:(j + 1) * BN, :] = blk.astype(jnp.bfloat16)
                    mv_ref[j, :] = mvb[0, :]

        @pl.when(i == GRID - 1)
        def _():
            mv = mv_ref[...]
            gidx = (lax.broadcasted_iota(jnp.int32, (GRID, BN), 0) * BN
                    + lax.broadcasted_iota(jnp.int32, (GRID, BN), 1))
            copies = []
            for t in range(PICK):
                m = jnp.max(mv)
                sel = jnp.min(jnp.where(mv >= m, gidx, IMAX))
                mv = jnp.where(gidx == sel, NEG, mv)
                cp = pltpu.make_async_copy(
                    embed_any.at[pl.ds(sel, 1), :],
                    rows2_ref.at[pl.ds(t, 1), :],
                    sem_row,
                )
                cp.start()
                copies.append(cp)
            for cp in copies:
                cp.wait()
            qr = lax.dot_general(rows2_ref[...], wq_ref[...], _DOT_NT,
                                 preferred_element_type=jnp.float32) + bqr_ref[...]
            bb = lax.dot_general(qr, wk_ref[...], _DOT_NN,
                                 preferred_element_type=jnp.float32)
            b16_ref[...] = bb.astype(jnp.bfloat16)
            dd = lax.dot_general(qr, bkc_ref[...], _DOT_NN,
                                 preferred_element_type=jnp.float32)
            rid = lax.broadcasted_iota(jnp.int32, (KPAD, 1), 0)
            d_ref[...] = jnp.where(rid >= PICK, NEG, dd)

        for j in range(GRID):
            @pl.when(i == GRID + j)
            def _(j=j):
                blkb = ebb_ref[j * BN:(j + 1) * BN, :]
                tt = lax.dot_general(b16_ref[...], blkb, _DOT_NT,
                                     preferred_element_type=jnp.float32)
                p = jnp.max(tt + d_ref[...], axis=0, keepdims=True)
                contrib = lax.dot_general(p.astype(jnp.bfloat16), blkb,
                                          _DOT_NN,
                                          preferred_element_type=jnp.float32)
                if j == 0:
                    out_ref[...] = contrib
                else:
                    out_ref[...] = out_ref[...] + contrib

    return pl.pallas_call(
        body,
        grid=(2 * GRID,),
        in_specs=[
            pl.BlockSpec((BN, D), lambda i: (jnp.minimum(i, GRID - 1), 0)),
            pl.BlockSpec(memory_space=pl.ANY),
            pl.BlockSpec((KPAD, D), lambda i: (0, 0)),
            pl.BlockSpec((D, D), lambda i: (0, 0)),
            pl.BlockSpec((D, D), lambda i: (0, 0)),
            pl.BlockSpec((D, 1), lambda i: (0, 0)),
            pl.BlockSpec((1, D), lambda i: (0, 0)),
            pl.BlockSpec((1, D), lambda i: (0, 0)),
            pl.BlockSpec((D, 1), lambda i: (0, 0)),
        ],
        out_specs=pl.BlockSpec((1, D), lambda i: (0, 0)),
        out_shape=jax.ShapeDtypeStruct((1, D), jnp.float32),
        scratch_shapes=[
            pltpu.VMEM((KPAD, D), jnp.float32),
            pltpu.VMEM((KPAD, 1), jnp.float32),
            pltpu.VMEM((KPAD, D), jnp.bfloat16),
            pltpu.VMEM((KPAD, 1), jnp.float32),
            pltpu.VMEM((GRID, BN), jnp.float32),
            pltpu.VMEM((N, D), jnp.bfloat16),
            pltpu.VMEM((KPAD, D), jnp.float32),
            pltpu.SemaphoreType.DMA,
        ],
    )(embed, embed, rows_s, Wq, Wk, bq_col, bq_row, bk_row, bk_col)


def kernel(embed_matrix, Wq, bq, Wk, bk, sample_indices):
    idx16 = jnp.concatenate(
        [sample_indices.astype(jnp.int32),
         jnp.zeros((KPAD - PICK,), jnp.int32)])
    rows_s = _sc_gather_rows(embed_matrix, idx16)
    return _fused_passes(embed_matrix, rows_s, Wq, Wk,
                         bq.reshape(D, 1), bq.reshape(1, D),
                         bk.reshape(1, D), bk.reshape(D, 1))


# speedup vs baseline: 1.5151x; 1.0510x over previous
import functools

import jax
import jax.numpy as jnp
from jax import lax
from jax.experimental import pallas as pl
from jax.experimental.pallas import tpu as pltpu
from jax.experimental.pallas import tpu_sc as plsc

N = 100000
D = 128
PICK = 12
KPAD = 16
BN = 20000
GRID = N // BN
NEG = -1e30
IMAX = 2147483647
_DOT_NT = (((1,), (1,)), ((), ()))
_DOT_NN = (((1,), (0,)), ((), ()))


def _sc_gather_rows(embed, idx16):
    mesh = plsc.VectorSubcoreMesh(core_axis_name="c", subcore_axis_name="s")

    @functools.partial(
        pl.kernel,
        out_type=jax.ShapeDtypeStruct((KPAD, D), jnp.float32),
        mesh=mesh,
        scratch_types=[
            pltpu.VMEM((KPAD,), jnp.int32),
            pltpu.VMEM((KPAD, D), jnp.float32),
            pltpu.SemaphoreType.DMA,
        ],
    )
    def gather_kernel(embed_hbm, idx_hbm, out_hbm, idx_v, rows_v, sem):
        c = lax.axis_index("c")
        s = lax.axis_index("s")

        @pl.when(jnp.logical_and(c == 0, s == 0))
        def _():
            pltpu.sync_copy(idx_hbm, idx_v)
            pltpu.async_copy(embed_hbm.at[idx_v], rows_v, sem).wait()
            pltpu.sync_copy(rows_v, out_hbm)

    return gather_kernel(embed, idx16)


def _fused_passes(embed, rows_s, Wq, Wk, bq_col, bq_row, bk_row, bk_col):
    def body(embed_ref, embed_any, rows_ref, wq_ref, wk_ref, bqc_ref,
             bqr_ref, bkr_ref, bkc_ref, out_ref, qa_ref, c_ref, b16_ref,
             d_ref, mv_ref, ebb_ref, rows2_ref, sem_row):
        i = pl.program_id(0)

        @pl.when(i == 0)
        def _():
            nk = lax.dot_general(rows_ref[...], wk_ref[...], _DOT_NT,
                                 preferred_element_type=jnp.float32) + bkr_ref[...]
            qa_ref[...] = lax.dot_general(nk, wq_ref[...], _DOT_NN,
                                          preferred_element_type=jnp.float32)
            cc = lax.dot_general(nk, bqc_ref[...], _DOT_NN,
                                 preferred_element_type=jnp.float32)
            rid = lax.broadcasted_iota(jnp.int32, (KPAD, 1), 0)
            c_ref[...] = jnp.where(rid >= PICK, NEG, cc)
            rows2_ref[...] = jnp.zeros((KPAD, D), jnp.float32)

        @pl.when(i < GRID)
        def _():
            blk = embed_ref[...]
            st = lax.dot_general(qa_ref[...], blk, _DOT_NT,
                                 preferred_element_type=jnp.float32)
            mvb = jnp.max(st + c_ref[...], axis=0, keepdims=True)
            for j in range(GRID):
                @pl.when(i == j)
                def _(blk=blk, mvb=mvb, j=j):
                    ebb_ref[j * BN:(j + 1) * BN, :] = blk.astype(jnp.bfloat16)
                    mv_ref[j, :] = mvb[0, :]

        @pl.when(i == GRID - 1)
        def _():
            mv = mv_ref[...]
            gidx = (lax.broadcasted_iota(jnp.int32, (GRID, BN), 0) * BN
                    + lax.broadcasted_iota(jnp.int32, (GRID, BN), 1))
            copies = []
            for t in range(PICK):
                m = jnp.max(mv)
                sel = jnp.min(jnp.where(mv >= m, gidx, IMAX))
                mv = jnp.where(gidx == sel, NEG, mv)
                cp = pltpu.make_async_copy(
                    embed_any.at[pl.ds(sel, 1), :],
                    rows2_ref.at[pl.ds(t, 1), :],
                    sem_row,
                )
                cp.start()
                copies.append(cp)
            for cp in copies:
                cp.wait()
            qr = lax.dot_general(rows2_ref[...], wq_ref[...], _DOT_NT,
                                 preferred_element_type=jnp.float32) + bqr_ref[...]
            bb = lax.dot_general(qr, wk_ref[...], _DOT_NN,
                                 preferred_element_type=jnp.float32)
            b16_ref[...] = bb.astype(jnp.bfloat16)
            dd = lax.dot_general(qr, bkc_ref[...], _DOT_NN,
                                 preferred_element_type=jnp.float32)
            rid = lax.broadcasted_iota(jnp.int32, (KPAD, 1), 0)
            d_ref[...] = jnp.where(rid >= PICK, NEG, dd)

        for j in range(GRID):
            @pl.when(i == GRID + j)
            def _(j=j):
                blkb = ebb_ref[j * BN:(j + 1) * BN, :]
                tt = lax.dot_general(b16_ref[...], blkb, _DOT_NT,
                                     preferred_element_type=jnp.float32)
                p = jnp.max(tt + d_ref[...], axis=0, keepdims=True)
                contrib = lax.dot_general(p.astype(jnp.bfloat16), blkb,
                                          _DOT_NN,
                                          preferred_element_type=jnp.float32)
                if j == 0:
                    out_ref[...] = contrib
                else:
                    out_ref[...] = out_ref[...] + contrib

    return pl.pallas_call(
        body,
        grid=(2 * GRID,),
        in_specs=[
            pl.BlockSpec((BN, D), lambda i: (jnp.minimum(i, GRID - 1), 0)),
            pl.BlockSpec(memory_space=pl.ANY),
            pl.BlockSpec((KPAD, D), lambda i: (0, 0)),
            pl.BlockSpec((D, D), lambda i: (0, 0)),
            pl.BlockSpec((D, D), lambda i: (0, 0)),
            pl.BlockSpec((D, 1), lambda i: (0, 0)),
            pl.BlockSpec((1, D), lambda i: (0, 0)),
            pl.BlockSpec((1, D), lambda i: (0, 0)),
            pl.BlockSpec((D, 1), lambda i: (0, 0)),
        ],
        out_specs=pl.BlockSpec((1, D), lambda i: (0, 0)),
        out_shape=jax.ShapeDtypeStruct((1, D), jnp.float32),
        scratch_shapes=[
            pltpu.VMEM((KPAD, D), jnp.float32),
            pltpu.VMEM((KPAD, 1), jnp.float32),
            pltpu.VMEM((KPAD, D), jnp.bfloat16),
            pltpu.VMEM((KPAD, 1), jnp.float32),
            pltpu.VMEM((GRID, BN), jnp.float32),
            pltpu.VMEM((N, D), jnp.bfloat16),
            pltpu.VMEM((KPAD, D), jnp.float32),
            pltpu.SemaphoreType.DMA,
        ],
    )(embed, embed, rows_s, Wq, Wk, bq_col, bq_row, bk_row, bk_col)


def kernel(embed_matrix, Wq, bq, Wk, bk, sample_indices):
    idx16 = jnp.concatenate(
        [sample_indices.astype(jnp.int32),
         jnp.zeros((KPAD - PICK,), jnp.int32)])
    rows_s = _sc_gather_rows(embed_matrix, idx16)
    return _fused_passes(embed_matrix, rows_s, Wq, Wk,
                         bq.reshape(D, 1), bq.reshape(1, D),
                         bk.reshape(1, D), bk.reshape(D, 1))
